# Initial kernel scaffold; baseline (speedup 1.0000x reference)
import jax
import jax.numpy as jnp
from jax.experimental import pallas as pl


def kernel(x, w_qkv, b_qkv, w_proj, b_proj):
    raise NotImplementedError("write your pallas kernel here")


# speedup vs baseline: 2.5048x; 2.5048x over previous
import functools

import jax
import jax.numpy as jnp
from jax import lax
from jax.experimental import pallas as pl
from jax.experimental.pallas import tpu as pltpu


def _qkv_kernel(x_ref, wq_ref, wk_ref, wv_ref, bq_ref, bk_ref, bv_ref,
                q_ref, k_ref, v_ref, *, q_scale):
    x = x_ref[...].astype(jnp.bfloat16)
    q = jnp.dot(x, wq_ref[...], preferred_element_type=jnp.float32)
    k = jnp.dot(x, wk_ref[...], preferred_element_type=jnp.float32)
    v = jnp.dot(x, wv_ref[...], preferred_element_type=jnp.float32)
    q_ref[...] = ((q + bq_ref[...]) * q_scale).astype(q_ref.dtype)
    k_ref[...] = (k + bk_ref[...]).astype(k_ref.dtype)
    v_ref[...] = (v + bv_ref[...]).astype(v_ref.dtype)


def _qkv_proj(x2, wq, wk, wv, bq, bk, bv, *, q_scale, tm=512):
    M, C = x2.shape
    assert M % tm == 0
    grid = (M // tm,)
    out_sds = jax.ShapeDtypeStruct((M, C), jnp.bfloat16)
    w_spec = pl.BlockSpec((C, C), lambda i: (0, 0))
    b_spec = pl.BlockSpec((1, C), lambda i: (0, 0))
    o_spec = pl.BlockSpec((tm, C), lambda i: (i, 0))
    return pl.pallas_call(
        functools.partial(_qkv_kernel, q_scale=q_scale),
        out_shape=(out_sds, out_sds, out_sds),
        grid=grid,
        in_specs=[
            pl.BlockSpec((tm, C), lambda i: (i, 0)),
            w_spec, w_spec, w_spec, b_spec, b_spec, b_spec,
        ],
        out_specs=(o_spec, o_spec, o_spec),
        compiler_params=pltpu.CompilerParams(
            dimension_semantics=("parallel",)),
        cost_estimate=pl.CostEstimate(
            flops=2 * M * 3 * C * C,
            transcendentals=0,
            bytes_accessed=M * C * 4 + 3 * C * C * 2 + 3 * M * C * 2),
    )(x2, wq, wk, wv, bq, bk, bv)


def _tri_qi(t, nblk):
    qi = jnp.int32(0)
    for j in range(1, nblk):
        qi = qi + (t >= j * (j + 1) // 2).astype(jnp.int32)
    return qi


def _attn_kernel(q_ref, k_ref, v_ref, wp_ref, bp_ref, o_ref,
                 m_scr, l_scr, acc_scr, *, n_head, hs, nblk, blk):
    t = pl.program_id(1)
    qi = _tri_qi(t, nblk)
    ki = t - qi * (qi + 1) // 2

    @pl.when(ki == 0)
    def _():
        m_scr[...] = jnp.full_like(m_scr, -jnp.inf)
        l_scr[...] = jnp.zeros_like(l_scr)
        acc_scr[...] = jnp.zeros_like(acc_scr)

    def process(masked):
        q = q_ref[0]
        k = k_ref[0]
        v = v_ref[0]
        m_prev = m_scr[...]
        l_prev = l_scr[...]
        acc_prev = acc_scr[...]

        if masked:
            row = lax.broadcasted_iota(jnp.int32, (blk, blk), 0)
            col = lax.broadcasted_iota(jnp.int32, (blk, blk), 1)
            causal = row >= col

        m_cols, l_cols, acc_cols = [], [], []
        for h in range(n_head):
            sl = slice(h * hs, (h + 1) * hs)
            s = lax.dot_general(q[:, sl], k[:, sl], (((1,), (1,)), ((), ())),
                                preferred_element_type=jnp.float32)
            if masked:
                s = jnp.where(causal, s, -1e30)
            m_h = m_prev[:, h:h + 1]
            m_new = jnp.maximum(m_h, s.max(axis=-1, keepdims=True))
            alpha = jnp.exp(m_h - m_new)
            p = jnp.exp(s - m_new)
            l_cols.append(alpha * l_prev[:, h:h + 1]
                          + p.sum(axis=-1, keepdims=True))
            pv = jnp.dot(p.astype(jnp.bfloat16), v[:, sl],
                         preferred_element_type=jnp.float32)
            acc_cols.append(alpha * acc_prev[:, sl] + pv)
            m_cols.append(m_new)

        m_scr[...] = jnp.concatenate(m_cols, axis=1)
        l_scr[...] = jnp.concatenate(l_cols, axis=1)
        acc_scr[...] = jnp.concatenate(acc_cols, axis=1)

    @pl.when(ki < qi)
    def _():
        process(False)

    @pl.when(ki == qi)
    def _():
        process(True)
        inv_l = pl.reciprocal(l_scr[...], approx=True)
        acc = acc_scr[...]
        y = jnp.concatenate(
            [acc[:, h * hs:(h + 1) * hs] * inv_l[:, h:h + 1]
             for h in range(n_head)], axis=1).astype(jnp.bfloat16)
        o_ref[0] = (jnp.dot(y, wp_ref[...], preferred_element_type=jnp.float32)
                    + bp_ref[...]).astype(o_ref.dtype)


def _flash_attn_proj(q, k, v, wp, bp, *, n_head, out_dtype, blk=256):
    B, T, C = q.shape
    assert T % blk == 0 and C % n_head == 0
    hs = C // n_head
    nblk = T // blk
    ntri = nblk * (nblk + 1) // 2

    def q_idx(b, t):
        return (b, _tri_qi(t, nblk), 0)

    def kv_idx(b, t):
        qi = _tri_qi(t, nblk)
        return (b, t - qi * (qi + 1) // 2, 0)

    kernel_fn = functools.partial(_attn_kernel, n_head=n_head, hs=hs,
                                  nblk=nblk, blk=blk)
    flops = 2 * B * C * T * (T + blk) + 2 * B * T * C * C
    return pl.pallas_call(
        kernel_fn,
        out_shape=jax.ShapeDtypeStruct((B, T, C), out_dtype),
        grid=(B, ntri),
        in_specs=[
            pl.BlockSpec((1, blk, C), q_idx),
            pl.BlockSpec((1, blk, C), kv_idx),
            pl.BlockSpec((1, blk, C), kv_idx),
            pl.BlockSpec((C, C), lambda b, t: (0, 0)),
            pl.BlockSpec((1, C), lambda b, t: (0, 0)),
        ],
        out_specs=pl.BlockSpec((1, blk, C), q_idx),
        scratch_shapes=[
            pltpu.VMEM((blk, n_head), jnp.float32),
            pltpu.VMEM((blk, n_head), jnp.float32),
            pltpu.VMEM((blk, C), jnp.float32),
        ],
        compiler_params=pltpu.CompilerParams(
            dimension_semantics=("parallel", "arbitrary")),
        cost_estimate=pl.CostEstimate(
            flops=flops,
            transcendentals=B * n_head * T * (T + blk) // 2,
            bytes_accessed=(3 * B * T * C * 2) * (nblk + 1) // 2
                           + B * T * C * 4 + C * C * 2),
    )(q, k, v, wp, bp)


def kernel(x, w_qkv, b_qkv, w_proj, b_proj):
    B, T, C = x.shape
    n_head = 12
    hs = C // n_head
    q_scale = 1.0 / (hs ** 0.5)

    wq = w_qkv[:, :C].astype(jnp.bfloat16)
    wk = w_qkv[:, C:2 * C].astype(jnp.bfloat16)
    wv = w_qkv[:, 2 * C:].astype(jnp.bfloat16)
    bq = b_qkv[:C].reshape(1, C)
    bk = b_qkv[C:2 * C].reshape(1, C)
    bv = b_qkv[2 * C:].reshape(1, C)

    x2 = x.reshape(B * T, C)
    q2, k2, v2 = _qkv_proj(x2, wq, wk, wv, bq, bk, bv, q_scale=q_scale)

    out = _flash_attn_proj(
        q2.reshape(B, T, C), k2.reshape(B, T, C), v2.reshape(B, T, C),
        w_proj.astype(jnp.bfloat16), b_proj.reshape(1, C),
        n_head=n_head, out_dtype=x.dtype)
    return out


# speedup vs baseline: 5.1953x; 2.0741x over previous
import functools

import jax
import jax.numpy as jnp
from jax import lax
from jax.experimental import pallas as pl
from jax.experimental.pallas import tpu as pltpu


def _qkv_kernel(x_ref, wq_ref, wk_ref, wv_ref, bq_ref, bk_ref, bv_ref,
                q_ref, k_ref, v_ref, *, q_scale):
    x = x_ref[...].astype(jnp.bfloat16)
    q = jnp.dot(x, wq_ref[...], preferred_element_type=jnp.float32)
    k = jnp.dot(x, wk_ref[...], preferred_element_type=jnp.float32)
    v = jnp.dot(x, wv_ref[...], preferred_element_type=jnp.float32)
    q_ref[...] = ((q + bq_ref[...]) * q_scale).astype(q_ref.dtype)
    k_ref[...] = (k + bk_ref[...]).astype(k_ref.dtype)
    v_ref[...] = (v + bv_ref[...]).astype(v_ref.dtype)


def _qkv_proj(x2, wq, wk, wv, bq, bk, bv, *, q_scale, tm=512):
    M, C = x2.shape
    assert M % tm == 0
    grid = (M // tm,)
    out_sds = jax.ShapeDtypeStruct((M, C), jnp.bfloat16)
    w_spec = pl.BlockSpec((C, C), lambda i: (0, 0))
    b_spec = pl.BlockSpec((1, C), lambda i: (0, 0))
    o_spec = pl.BlockSpec((tm, C), lambda i: (i, 0))
    return pl.pallas_call(
        functools.partial(_qkv_kernel, q_scale=q_scale),
        out_shape=(out_sds, out_sds, out_sds),
        grid=grid,
        in_specs=[
            pl.BlockSpec((tm, C), lambda i: (i, 0)),
            w_spec, w_spec, w_spec, b_spec, b_spec, b_spec,
        ],
        out_specs=(o_spec, o_spec, o_spec),
        compiler_params=pltpu.CompilerParams(
            dimension_semantics=("parallel",)),
        cost_estimate=pl.CostEstimate(
            flops=2 * M * 3 * C * C,
            transcendentals=0,
            bytes_accessed=M * C * 4 + 3 * C * C * 2 + 3 * M * C * 2),
    )(x2, wq, wk, wv, bq, bk, bv)


def _attn_kernel(q_ref, k_ref, v_ref, wp_ref, bp_ref, o_ref,
                 *, n_head, hs, blk, T):
    qi = pl.program_id(1)
    q = q_ref[0]
    k = k_ref[0]
    v = v_ref[0]

    row = qi * blk + lax.broadcasted_iota(jnp.int32, (blk, T), 0)
    col = lax.broadcasted_iota(jnp.int32, (blk, T), 1)
    bias = jnp.where(row >= col, 0.0, -1e30)

    y_cols = []
    for h in range(n_head):
        sl = slice(h * hs, (h + 1) * hs)
        s = lax.dot_general(q[:, sl], k[:, sl], (((1,), (1,)), ((), ())),
                            preferred_element_type=jnp.float32)
        s = s + bias
        m = s.max(axis=-1, keepdims=True)
        p = jnp.exp(s - m)
        l = p.sum(axis=-1, keepdims=True)
        pv = jnp.dot(p.astype(jnp.bfloat16), v[:, sl],
                     preferred_element_type=jnp.float32)
        y_cols.append(pv * pl.reciprocal(l, approx=True))
    y = jnp.concatenate(y_cols, axis=1).astype(jnp.bfloat16)
    o_ref[0] = (jnp.dot(y, wp_ref[...], preferred_element_type=jnp.float32)
                + bp_ref[...]).astype(o_ref.dtype)


def _flash_attn_proj(q, k, v, wp, bp, *, n_head, out_dtype, blk=256):
    B, T, C = q.shape
    assert T % blk == 0 and C % n_head == 0
    hs = C // n_head
    nblk = T // blk

    kernel_fn = functools.partial(_attn_kernel, n_head=n_head, hs=hs,
                                  blk=blk, T=T)
    flops = 2 * B * C * T * T * 2 + 2 * B * T * C * C
    return pl.pallas_call(
        kernel_fn,
        out_shape=jax.ShapeDtypeStruct((B, T, C), out_dtype),
        grid=(B, nblk),
        in_specs=[
            pl.BlockSpec((1, blk, C), lambda b, i: (b, i, 0)),
            pl.BlockSpec((1, T, C), lambda b, i: (b, 0, 0)),
            pl.BlockSpec((1, T, C), lambda b, i: (b, 0, 0)),
            pl.BlockSpec((C, C), lambda b, i: (0, 0)),
            pl.BlockSpec((1, C), lambda b, i: (0, 0)),
        ],
        out_specs=pl.BlockSpec((1, blk, C), lambda b, i: (b, i, 0)),
        compiler_params=pltpu.CompilerParams(
            dimension_semantics=("parallel", "arbitrary")),
        cost_estimate=pl.CostEstimate(
            flops=flops,
            transcendentals=B * n_head * T * T,
            bytes_accessed=3 * B * T * C * 2 + B * T * C * 4 + C * C * 2),
    )(q, k, v, wp, bp)


def kernel(x, w_qkv, b_qkv, w_proj, b_proj):
    B, T, C = x.shape
    n_head = 12
    hs = C // n_head
    q_scale = 1.0 / (hs ** 0.5)

    wq = w_qkv[:, :C].astype(jnp.bfloat16)
    wk = w_qkv[:, C:2 * C].astype(jnp.bfloat16)
    wv = w_qkv[:, 2 * C:].astype(jnp.bfloat16)
    bq = b_qkv[:C].reshape(1, C)
    bk = b_qkv[C:2 * C].reshape(1, C)
    bv = b_qkv[2 * C:].reshape(1, C)

    x2 = x.reshape(B * T, C)
    q2, k2, v2 = _qkv_proj(x2, wq, wk, wv, bq, bk, bv, q_scale=q_scale)

    out = _flash_attn_proj(
        q2.reshape(B, T, C), k2.reshape(B, T, C), v2.reshape(B, T, C),
        w_proj.astype(jnp.bfloat16), b_proj.reshape(1, C),
        n_head=n_head, out_dtype=x.dtype)
    return out
